# baseline (device time: 83968 ns/iter reference)
import jax
import jax.numpy as jnp
from jax import lax
from jax.experimental import pallas as pl
from jax.experimental.pallas import tpu as pltpu

N_DEV = 4


def kernel(x, router_W, route_idx, expert_W):
    m, d = x.shape
    e_loc, _, h = expert_W.shape
    n_exp = N_DEV * e_loc

    def body(x_ref, rw_ref, idx_ref, ew_ref, out_ref, comm_ref, send_sems, recv_sems):
        my = lax.axis_index("i")
        left = lax.rem(my + N_DEV - 1, N_DEV)
        right = lax.rem(my + 1, N_DEV)

        barrier_sem = pltpu.get_barrier_semaphore()
        for nbr in (left, right):
            pl.semaphore_signal(
                barrier_sem,
                inc=1,
                device_id=(nbr,),
                device_id_type=pl.DeviceIdType.MESH,
            )
        pl.semaphore_wait(barrier_sem, 2)

        rdma = pltpu.make_async_remote_copy(
            src_ref=ew_ref,
            dst_ref=comm_ref.at[0],
            send_sem=send_sems.at[0],
            recv_sem=recv_sems.at[0],
            device_id=(right,),
            device_id_type=pl.DeviceIdType.MESH,
        )
        rdma.start()

        xv = x_ref[:, :]
        scores = jnp.dot(xv, rw_ref[:, :], preferred_element_type=jnp.float32)
        p = jnp.exp(scores - jnp.max(scores, axis=-1, keepdims=True))
        p = p / jnp.sum(p, axis=-1, keepdims=True)
        eids = lax.broadcasted_iota(jnp.int32, (m, n_exp), 1)
        i0 = idx_ref[:, 0:1]
        i1 = idx_ref[:, 1:2]
        p0 = jnp.sum(jnp.where(eids == i0, p, 0.0), axis=-1, keepdims=True)
        p1 = jnp.sum(jnp.where(eids == i1, p, 0.0), axis=-1, keepdims=True)
        g0 = p0 / (p0 + p1)
        g1 = p1 / (p0 + p1)

        def shard_contrib(s, w_ref):
            acc = jnp.zeros((m, h), dtype=jnp.float32)
            for k in range(e_loc):
                g = s * e_loc + k
                gate = jnp.where(i0 == g, g0, 0.0) + jnp.where(i1 == g, g1, 0.0)
                acc = acc + jnp.dot(
                    xv * gate, w_ref[k], preferred_element_type=jnp.float32
                )
            return acc

        acc = shard_contrib(my, ew_ref)
        rdma.wait()

        for hop in range(1, N_DEV - 1):
            rdma = pltpu.make_async_remote_copy(
                src_ref=comm_ref.at[hop - 1],
                dst_ref=comm_ref.at[hop],
                send_sem=send_sems.at[hop],
                recv_sem=recv_sems.at[hop],
                device_id=(right,),
                device_id_type=pl.DeviceIdType.MESH,
            )
            rdma.start()
            acc = acc + shard_contrib(
                lax.rem(my - hop + N_DEV, N_DEV), comm_ref.at[hop - 1]
            )
            rdma.wait()
        acc = acc + shard_contrib(
            lax.rem(my + 1, N_DEV), comm_ref.at[N_DEV - 2]
        )
        out_ref[:, :] = acc

    return pl.pallas_call(
        body,
        out_shape=jax.ShapeDtypeStruct((m, h), jnp.float32),
        in_specs=[pl.BlockSpec(memory_space=pltpu.VMEM)] * 4,
        out_specs=pl.BlockSpec(memory_space=pltpu.VMEM),
        scratch_shapes=[
            pltpu.VMEM((N_DEV - 1, e_loc, d, h), jnp.float32),
            pltpu.SemaphoreType.DMA((N_DEV - 1,)),
            pltpu.SemaphoreType.DMA((N_DEV - 1,)),
        ],
        compiler_params=pltpu.CompilerParams(collective_id=0),
    )(x, router_W, route_idx, expert_W)


# device time: 49629 ns/iter; 1.6919x vs baseline; 1.6919x over previous
import jax
import jax.numpy as jnp
from jax import lax
from jax.experimental import pallas as pl
from jax.experimental.pallas import tpu as pltpu

N_DEV = 4


def kernel(x, router_W, route_idx, expert_W):
    m, d = x.shape
    e_loc, _, h = expert_W.shape
    n_exp = N_DEV * e_loc
    half = e_loc // 2

    def body(
        x_ref, rw_ref, idx_ref, ew_ref, out_ref,
        comm_r, comm_l, send_r, recv_r, send_l, recv_l,
    ):
        my = lax.axis_index("i")
        left = lax.rem(my + N_DEV - 1, N_DEV)
        right = lax.rem(my + 1, N_DEV)

        barrier_sem = pltpu.get_barrier_semaphore()
        for nbr in (left, right):
            pl.semaphore_signal(
                barrier_sem,
                inc=1,
                device_id=(nbr,),
                device_id_type=pl.DeviceIdType.MESH,
            )
        pl.semaphore_wait(barrier_sem, 2)

        def mk(src, dst, ssem, rsem, dev):
            return pltpu.make_async_remote_copy(
                src_ref=src,
                dst_ref=dst,
                send_sem=ssem,
                recv_sem=rsem,
                device_id=(dev,),
                device_id_type=pl.DeviceIdType.MESH,
            )

        r = mk(ew_ref.at[pl.ds(0, half)], comm_r.at[0],
               send_r.at[0], recv_r.at[0], right)
        l = mk(ew_ref.at[pl.ds(half, half)], comm_l.at[0],
               send_l.at[0], recv_l.at[0], left)
        r.start()
        l.start()

        xv = x_ref[:, :]
        scores = jnp.dot(xv, rw_ref[:, :], preferred_element_type=jnp.float32)
        p = jnp.exp(scores - jnp.max(scores, axis=-1, keepdims=True))
        p = p / jnp.sum(p, axis=-1, keepdims=True)
        eids = lax.broadcasted_iota(jnp.int32, (m, n_exp), 1)
        i0 = idx_ref[:, 0:1]
        i1 = idx_ref[:, 1:2]
        p0 = jnp.sum(jnp.where(eids == i0, p, 0.0), axis=-1, keepdims=True)
        p1 = jnp.sum(jnp.where(eids == i1, p, 0.0), axis=-1, keepdims=True)
        g0 = p0 / (p0 + p1)
        g1 = p1 / (p0 + p1)

        def contrib(e_base, w_ref, n, acc):
            for k in range(n):
                g = e_base + k
                gate = jnp.where(i0 == g, g0, 0.0) + jnp.where(i1 == g, g1, 0.0)
                acc = acc + jnp.dot(
                    xv * gate, w_ref[k], preferred_element_type=jnp.float32
                )
            return acc

        acc = contrib(my * e_loc, ew_ref, e_loc,
                      jnp.zeros((m, h), dtype=jnp.float32))
        r.wait()
        l.wait()

        for j in range(N_DEV - 2):
            r = mk(comm_r.at[j], comm_r.at[j + 1],
                   send_r.at[j + 1], recv_r.at[j + 1], right)
            l = mk(comm_l.at[j], comm_l.at[j + 1],
                   send_l.at[j + 1], recv_l.at[j + 1], left)
            r.start()
            l.start()
            sr = lax.rem(my - (j + 1) + N_DEV, N_DEV)
            sl = lax.rem(my + (j + 1), N_DEV)
            acc = contrib(sr * e_loc, comm_r.at[j], half, acc)
            acc = contrib(sl * e_loc + half, comm_l.at[j], half, acc)
            r.wait()
            l.wait()

        sr = lax.rem(my + 1, N_DEV)
        sl = lax.rem(my + N_DEV - 1, N_DEV)
        acc = contrib(sr * e_loc, comm_r.at[N_DEV - 2], half, acc)
        acc = contrib(sl * e_loc + half, comm_l.at[N_DEV - 2], half, acc)
        out_ref[:, :] = acc

    return pl.pallas_call(
        body,
        out_shape=jax.ShapeDtypeStruct((m, h), jnp.float32),
        in_specs=[pl.BlockSpec(memory_space=pltpu.VMEM)] * 4,
        out_specs=pl.BlockSpec(memory_space=pltpu.VMEM),
        scratch_shapes=[
            pltpu.VMEM((N_DEV - 1, half, d, h), jnp.float32),
            pltpu.VMEM((N_DEV - 1, half, d, h), jnp.float32),
            pltpu.SemaphoreType.DMA((N_DEV - 1,)),
            pltpu.SemaphoreType.DMA((N_DEV - 1,)),
            pltpu.SemaphoreType.DMA((N_DEV - 1,)),
            pltpu.SemaphoreType.DMA((N_DEV - 1,)),
        ],
        compiler_params=pltpu.CompilerParams(collective_id=0),
    )(x, router_W, route_idx, expert_W)


# device time: 32989 ns/iter; 2.5453x vs baseline; 1.5044x over previous
import jax
import jax.numpy as jnp
from jax import lax
from jax.experimental import pallas as pl
from jax.experimental.pallas import tpu as pltpu

N_DEV = 4


def kernel(x, router_W, route_idx, expert_W):
    m, d = x.shape
    e_loc, _, h = expert_W.shape
    n_exp = N_DEV * e_loc
    half = e_loc // 2

    def body(
        x_ref, rw_ref, idx_ref, ew_ref, out_ref,
        ew16_ref, comm_r, comm_l, send_r, recv_r, send_l, recv_l,
    ):
        my = lax.axis_index("i")
        left = lax.rem(my + N_DEV - 1, N_DEV)
        right = lax.rem(my + 1, N_DEV)

        barrier_sem = pltpu.get_barrier_semaphore()
        for nbr in (left, right):
            pl.semaphore_signal(
                barrier_sem,
                inc=1,
                device_id=(nbr,),
                device_id_type=pl.DeviceIdType.MESH,
            )
        ew16_ref[:, :, :] = ew_ref[:, :, :].astype(jnp.bfloat16)
        pl.semaphore_wait(barrier_sem, 2)

        def mk(src, dst, ssem, rsem, dev):
            return pltpu.make_async_remote_copy(
                src_ref=src,
                dst_ref=dst,
                send_sem=ssem,
                recv_sem=rsem,
                device_id=(dev,),
                device_id_type=pl.DeviceIdType.MESH,
            )

        r = mk(ew16_ref.at[pl.ds(0, half)], comm_r.at[0],
               send_r.at[0], recv_r.at[0], right)
        l = mk(ew16_ref.at[pl.ds(half, half)], comm_l.at[0],
               send_l.at[0], recv_l.at[0], left)
        r.start()
        l.start()

        xv = x_ref[:, :]
        scores = jnp.dot(xv, rw_ref[:, :], preferred_element_type=jnp.float32)
        p = jnp.exp(scores - jnp.max(scores, axis=-1, keepdims=True))
        p = p / jnp.sum(p, axis=-1, keepdims=True)
        eids = lax.broadcasted_iota(jnp.int32, (m, n_exp), 1)
        i0 = idx_ref[:, 0:1]
        i1 = idx_ref[:, 1:2]
        p0 = jnp.sum(jnp.where(eids == i0, p, 0.0), axis=-1, keepdims=True)
        p1 = jnp.sum(jnp.where(eids == i1, p, 0.0), axis=-1, keepdims=True)
        g0 = p0 / (p0 + p1)
        g1 = p1 / (p0 + p1)

        def contrib(e_base, w_ref, n, acc):
            for k in range(n):
                g = e_base + k
                gate = jnp.where(i0 == g, g0, 0.0) + jnp.where(i1 == g, g1, 0.0)
                xg = (xv * gate).astype(jnp.bfloat16)
                acc = acc + jnp.dot(
                    xg, w_ref[k], preferred_element_type=jnp.float32
                )
            return acc

        acc = contrib(my * e_loc, ew16_ref, e_loc,
                      jnp.zeros((m, h), dtype=jnp.float32))
        r.wait()
        l.wait()

        for j in range(N_DEV - 2):
            r = mk(comm_r.at[j], comm_r.at[j + 1],
                   send_r.at[j + 1], recv_r.at[j + 1], right)
            l = mk(comm_l.at[j], comm_l.at[j + 1],
                   send_l.at[j + 1], recv_l.at[j + 1], left)
            r.start()
            l.start()
            sr = lax.rem(my - (j + 1) + N_DEV, N_DEV)
            sl = lax.rem(my + (j + 1), N_DEV)
            acc = contrib(sr * e_loc, comm_r.at[j], half, acc)
            acc = contrib(sl * e_loc + half, comm_l.at[j], half, acc)
            r.wait()
            l.wait()

        sr = lax.rem(my + 1, N_DEV)
        sl = lax.rem(my + N_DEV - 1, N_DEV)
        acc = contrib(sr * e_loc, comm_r.at[N_DEV - 2], half, acc)
        acc = contrib(sl * e_loc + half, comm_l.at[N_DEV - 2], half, acc)
        out_ref[:, :] = acc

    return pl.pallas_call(
        body,
        out_shape=jax.ShapeDtypeStruct((m, h), jnp.float32),
        in_specs=[pl.BlockSpec(memory_space=pltpu.VMEM)] * 4,
        out_specs=pl.BlockSpec(memory_space=pltpu.VMEM),
        scratch_shapes=[
            pltpu.VMEM((e_loc, d, h), jnp.bfloat16),
            pltpu.VMEM((N_DEV - 1, half, d, h), jnp.bfloat16),
            pltpu.VMEM((N_DEV - 1, half, d, h), jnp.bfloat16),
            pltpu.SemaphoreType.DMA((N_DEV - 1,)),
            pltpu.SemaphoreType.DMA((N_DEV - 1,)),
            pltpu.SemaphoreType.DMA((N_DEV - 1,)),
            pltpu.SemaphoreType.DMA((N_DEV - 1,)),
        ],
        compiler_params=pltpu.CompilerParams(collective_id=0),
    )(x, router_W, route_idx, expert_W)


# device time: 30964 ns/iter; 2.7118x vs baseline; 1.0654x over previous
import jax
import jax.numpy as jnp
from jax import lax
from jax.experimental import pallas as pl
from jax.experimental.pallas import tpu as pltpu

N_DEV = 4


def kernel(x, router_W, route_idx, expert_W):
    m, d = x.shape
    e_loc, _, h = expert_W.shape
    n_exp = N_DEV * e_loc
    half = e_loc // 2

    def body(
        x_ref, rw_ref, idx_ref, ew_ref, out_ref,
        ew16_ref, full_r, full_l, diag_lo, diag_hi,
        sem_a, sem_b, sem_c, sem_d,
    ):
        my = lax.axis_index("i")
        left = lax.rem(my + N_DEV - 1, N_DEV)
        right = lax.rem(my + 1, N_DEV)
        diag = lax.rem(my + 2, N_DEV)

        barrier_sem = pltpu.get_barrier_semaphore()
        for nbr in (left, right):
            pl.semaphore_signal(
                barrier_sem,
                inc=1,
                device_id=(nbr,),
                device_id_type=pl.DeviceIdType.MESH,
            )
        ew16_ref[:, :, :] = ew_ref[:, :, :].astype(jnp.bfloat16)
        pl.semaphore_wait(barrier_sem, 2)

        def mk(src, dst, sems, dev):
            return pltpu.make_async_remote_copy(
                src_ref=src,
                dst_ref=dst,
                send_sem=sems.at[0],
                recv_sem=sems.at[1],
                device_id=(dev,),
                device_id_type=pl.DeviceIdType.MESH,
            )

        rdma_a = mk(ew16_ref, full_r, sem_a, left)
        rdma_b = mk(ew16_ref, full_l, sem_b, right)
        rdma_a.start()
        rdma_b.start()

        xv = x_ref[:, :]
        scores = jnp.dot(xv, rw_ref[:, :], preferred_element_type=jnp.float32)
        p = jnp.exp(scores - jnp.max(scores, axis=-1, keepdims=True))
        p = p / jnp.sum(p, axis=-1, keepdims=True)
        eids = lax.broadcasted_iota(jnp.int32, (m, n_exp), 1)
        i0 = idx_ref[:, 0:1]
        i1 = idx_ref[:, 1:2]
        p0 = jnp.sum(jnp.where(eids == i0, p, 0.0), axis=-1, keepdims=True)
        p1 = jnp.sum(jnp.where(eids == i1, p, 0.0), axis=-1, keepdims=True)
        g0 = p0 / (p0 + p1)
        g1 = p1 / (p0 + p1)

        def contrib(e_base, w_ref, n, acc):
            for k in range(n):
                g = e_base + k
                gate = jnp.where(i0 == g, g0, 0.0) + jnp.where(i1 == g, g1, 0.0)
                xg = (xv * gate).astype(jnp.bfloat16)
                acc = acc + jnp.dot(
                    xg, w_ref[k], preferred_element_type=jnp.float32
                )
            return acc

        acc = contrib(my * e_loc, ew16_ref, e_loc,
                      jnp.zeros((m, h), dtype=jnp.float32))

        rdma_a.wait_recv()
        rdma_c = mk(full_r.at[pl.ds(0, half)], diag_lo, sem_c, left)
        rdma_c.start()
        rdma_b.wait_recv()
        rdma_d = mk(full_l.at[pl.ds(half, half)], diag_hi, sem_d, right)
        rdma_d.start()

        acc = contrib(right * e_loc, full_r, e_loc, acc)
        acc = contrib(left * e_loc, full_l, e_loc, acc)

        rdma_c.wait_recv()
        acc = contrib(diag * e_loc, diag_lo, half, acc)
        rdma_d.wait_recv()
        acc = contrib(diag * e_loc + half, diag_hi, half, acc)
        out_ref[:, :] = acc

        rdma_a.wait_send()
        rdma_b.wait_send()
        rdma_c.wait_send()
        rdma_d.wait_send()

    return pl.pallas_call(
        body,
        out_shape=jax.ShapeDtypeStruct((m, h), jnp.float32),
        in_specs=[pl.BlockSpec(memory_space=pltpu.VMEM)] * 4,
        out_specs=pl.BlockSpec(memory_space=pltpu.VMEM),
        scratch_shapes=[
            pltpu.VMEM((e_loc, d, h), jnp.bfloat16),
            pltpu.VMEM((e_loc, d, h), jnp.bfloat16),
            pltpu.VMEM((e_loc, d, h), jnp.bfloat16),
            pltpu.VMEM((half, d, h), jnp.bfloat16),
            pltpu.VMEM((half, d, h), jnp.bfloat16),
            pltpu.SemaphoreType.DMA((2,)),
            pltpu.SemaphoreType.DMA((2,)),
            pltpu.SemaphoreType.DMA((2,)),
            pltpu.SemaphoreType.DMA((2,)),
        ],
        compiler_params=pltpu.CompilerParams(collective_id=0),
    )(x, router_W, route_idx, expert_W)
